# baseline (speedup 1.0000x reference)
import functools

import jax
import jax.numpy as jnp
from jax import lax
from jax.experimental import pallas as pl
from jax.experimental.pallas import tpu as pltpu
from jax.experimental.pallas import tpu_sc as plsc

BATCH = 4096
HIST = 50
DIM = 64
TOTAL = BATCH * HIST
NUM_WORKERS = 32
PER_WORKER = TOTAL // NUM_WORKERS
CHUNK = 128
NCHUNK = PER_WORKER // CHUNK

_mesh = plsc.VectorSubcoreMesh(core_axis_name="c", subcore_axis_name="s")


@functools.partial(
    pl.kernel,
    mesh=_mesh,
    out_type=jax.ShapeDtypeStruct((NUM_WORKERS, NCHUNK, CHUNK, DIM), jnp.float32),
    scratch_types=[
        pltpu.VMEM((NCHUNK, CHUNK), jnp.int32),
        pltpu.VMEM((CHUNK, DIM), jnp.float32),
        pltpu.SemaphoreType.DMA,
    ],
    compiler_params=pltpu.CompilerParams(use_tc_tiling_on_sc=False),
)
def _emb_lookup(idx_hbm, table_hbm, out_hbm, idx_v, rows_v, gsem):
    wid = lax.axis_index("s") * 2 + lax.axis_index("c")
    pltpu.sync_copy(idx_hbm.at[wid], idx_v)

    def step(j, carry):
        pltpu.async_copy(table_hbm.at[idx_v.at[j]], rows_v, gsem).wait()
        pltpu.sync_copy(rows_v, out_hbm.at[wid, j])
        return carry

    lax.fori_loop(0, NCHUNK, step, 0)


def kernel(inputs, weight):
    idx = inputs.reshape(NUM_WORKERS, NCHUNK, CHUNK).astype(jnp.int32)
    out = _emb_lookup(idx, weight)
    return out.reshape(BATCH, HIST, DIM)


# speedup vs baseline: 1.0443x; 1.0443x over previous
import functools

import jax
import jax.numpy as jnp
from jax import lax
from jax.experimental import pallas as pl
from jax.experimental.pallas import tpu as pltpu
from jax.experimental.pallas import tpu_sc as plsc

BATCH = 4096
HIST = 50
DIM = 64
TOTAL = BATCH * HIST
NUM_WORKERS = 32
PER_WORKER = TOTAL // NUM_WORKERS
CHUNK = 128
NCHUNK = PER_WORKER // CHUNK
NBUF = 5
NGRP = NCHUNK // NBUF

_mesh = plsc.VectorSubcoreMesh(core_axis_name="c", subcore_axis_name="s")


@functools.partial(
    pl.kernel,
    mesh=_mesh,
    out_type=jax.ShapeDtypeStruct((NUM_WORKERS, NCHUNK, CHUNK, DIM), jnp.float32),
    scratch_types=[
        pltpu.VMEM((NCHUNK, CHUNK), jnp.int32),
        pltpu.VMEM((NBUF, CHUNK, DIM), jnp.float32),
    ] + [pltpu.SemaphoreType.DMA] * NBUF,
    compiler_params=pltpu.CompilerParams(use_tc_tiling_on_sc=False),
)
def _emb_lookup(idx_hbm, table_hbm, out_hbm, idx_v, rows_v, *sems):
    wid = lax.axis_index("s") * 2 + lax.axis_index("c")
    pltpu.sync_copy(idx_hbm.at[wid], idx_v)

    for b in range(NBUF):
        pltpu.async_copy(table_hbm.at[idx_v.at[b]], rows_v.at[b], sems[b])

    def grp(g, carry):
        for b in range(NBUF):
            j = g * NBUF + b
            pltpu.make_async_copy(out_hbm.at[0, 0], rows_v.at[b], sems[b]).wait()
            pltpu.sync_copy(rows_v.at[b], out_hbm.at[wid, j])
            pltpu.async_copy(table_hbm.at[idx_v.at[j + NBUF]], rows_v.at[b], sems[b])
        return carry

    lax.fori_loop(0, NGRP - 1, grp, 0)

    for b in range(NBUF):
        j = (NGRP - 1) * NBUF + b
        pltpu.make_async_copy(out_hbm.at[0, 0], rows_v.at[b], sems[b]).wait()
        pltpu.sync_copy(rows_v.at[b], out_hbm.at[wid, j])


def kernel(inputs, weight):
    idx = inputs.reshape(NUM_WORKERS, NCHUNK, CHUNK).astype(jnp.int32)
    out = _emb_lookup(idx, weight)
    return out.reshape(BATCH, HIST, DIM)
